# Initial kernel scaffold; baseline (speedup 1.0000x reference)
import jax
import jax.numpy as jnp
from jax.experimental import pallas as pl


def kernel(collision_idxs, faces_segm, faces_parents, extra_pairs):
    raise NotImplementedError("write your pallas kernel here")


# speedup vs baseline: 5.5652x; 5.5652x over previous
import functools

import jax
import jax.numpy as jnp
from jax import lax
from jax.experimental import pallas as pl
from jax.experimental.pallas import tpu as pltpu
from jax.experimental.pallas import tpu_sc as plsc

_L = 16
_NW = 32
_GROUP = 2 * _L
_NFACE = 128


@functools.lru_cache(maxsize=None)
def _build_sc_kernel(n_vals: int, n_extra: int):
    assert n_vals % _GROUP == 0
    groups = n_vals // _GROUP
    gpw = -(-groups // _NW)
    gpw = -(-gpw // 4) * 4
    chunk = gpw * _GROUP
    mesh = plsc.VectorSubcoreMesh(core_axis_name="c", subcore_axis_name="s")

    @functools.partial(
        pl.kernel,
        mesh=mesh,
        out_type=jax.ShapeDtypeStruct((n_vals,), jnp.int32),
        compiler_params=pltpu.CompilerParams(needs_layout_passes=False),
        scratch_types=[
            pltpu.VMEM((chunk,), jnp.int32),
            pltpu.VMEM((chunk,), jnp.int32),
            pltpu.VMEM((_NFACE,), jnp.int32),
            pltpu.VMEM((_NFACE,), jnp.int32),
            pltpu.VMEM((2 * n_extra * _L,), jnp.int32),
            pltpu.VMEM((_NFACE,), jnp.int32),
            pltpu.VMEM((_NFACE,), jnp.int32),
        ],
    )
    def sc_kernel(ci_hbm, segm_hbm, par_hbm, ep_hbm, out_hbm,
                  in_v, out_v, segm_v, par_v, ep_v, st_v, ef_v):
        wid = lax.axis_index("s") * jnp.int32(2) + lax.axis_index("c")
        start_g = jnp.minimum(wid * jnp.int32(gpw), jnp.int32(groups - gpw))
        base = start_g * jnp.int32(_GROUP)
        pltpu.sync_copy(ci_hbm.at[pl.ds(base, chunk)], in_v)
        pltpu.sync_copy(segm_hbm, segm_v)
        pltpu.sync_copy(par_hbm, par_v)
        pltpu.sync_copy(ep_hbm, ep_v)

        lane2 = lax.iota(jnp.int32, _L) * jnp.int32(2)
        one = jnp.full((_L,), 1, jnp.int32)
        zero = jnp.full((_L,), 0, jnp.int32)
        neg1 = jnp.full((_L,), -1, jnp.int32)

        bits = []
        for k in range(n_extra):
            a = ep_v[pl.ds(2 * k * _L, _L)]
            b = ep_v[pl.ds((2 * k + 1) * _L, _L)]
            bits.append((a, b, one << a, one << b))

        for j in range(_NFACE // _L):
            sl = pl.ds(j * _L, _L)
            s = segm_v[sl]
            p = par_v[sl]
            st_v[sl] = s | (p << 7)
            ef = zero
            for a, b, bit_a, bit_b in bits:
                ef = ef | jnp.where(s == a, bit_b, zero)
                ef = ef | jnp.where(s == b, bit_a, zero)
            ef_v[sl] = ef

        @plsc.parallel_loop(jnp.int32(0), jnp.int32(chunk), jnp.int32(_GROUP),
                            unroll=4)
        def _loop(off):
            idx_a = lane2 + off
            idx_b = idx_a + jnp.int32(1)
            ia = plsc.load_gather(in_v, [idx_a])
            ib = plsc.load_gather(in_v, [idx_b])
            sta = plsc.load_gather(st_v, [ia])
            stb = plsc.load_gather(st_v, [ib])
            efa = plsc.load_gather(ef_v, [ia])
            s0 = sta & jnp.int32(127)
            p0 = sta >> jnp.int32(7)
            s1 = stb & jnp.int32(127)
            p1 = stb >> jnp.int32(7)
            t = ((s0 == s1) | (s0 == p1)) | (s1 == p0)
            t = t & (ia >= jnp.int32(0))
            m = t | (((efa >> s1) & one) != zero)
            plsc.store_scatter(out_v, [idx_a], jnp.where(m, neg1, ia))
            plsc.store_scatter(out_v, [idx_b], jnp.where(m, neg1, ib))

        pltpu.sync_copy(out_v, out_hbm.at[pl.ds(base, chunk)])

    return sc_kernel


def kernel(collision_idxs, faces_segm, faces_parents, extra_pairs):
    shape = collision_idxs.shape
    dt = collision_idxs.dtype
    flat = collision_idxs.astype(jnp.int32).reshape(-1)
    segm = faces_segm.astype(jnp.int32)
    par = faces_parents.astype(jnp.int32)
    n_extra = extra_pairs.shape[0]
    ep = jnp.repeat(extra_pairs.astype(jnp.int32).reshape(-1), _L)
    out = _build_sc_kernel(flat.shape[0], n_extra)(flat, segm, par, ep)
    return out.reshape(shape).astype(dt)


# speedup vs baseline: 9.4618x; 1.7002x over previous
import functools

import jax
import jax.numpy as jnp
from jax import lax
from jax.experimental import pallas as pl
from jax.experimental.pallas import tpu as pltpu
from jax.experimental.pallas import tpu_sc as plsc

_L = 16
_NW = 32
_GROUPW = 4 * _L
_NFACE = 128
_UNROLL = 2


@functools.lru_cache(maxsize=None)
def _build_sc_kernel(n_words: int, n_extra: int):
    assert n_words % _GROUPW == 0
    groups = n_words // _GROUPW
    gpw = -(-groups // _NW)
    gpw = -(-gpw // _UNROLL) * _UNROLL
    chunk = gpw * _GROUPW
    mesh = plsc.VectorSubcoreMesh(core_axis_name="c", subcore_axis_name="s")

    @functools.partial(
        pl.kernel,
        mesh=mesh,
        out_type=jax.ShapeDtypeStruct((n_words,), jnp.int32),
        compiler_params=pltpu.CompilerParams(needs_layout_passes=False),
        scratch_types=[
            pltpu.VMEM((chunk,), jnp.int32),
            pltpu.VMEM((chunk,), jnp.int32),
            pltpu.VMEM((_NFACE,), jnp.int32),
            pltpu.VMEM((_NFACE,), jnp.int32),
            pltpu.VMEM((2 * n_extra * _L,), jnp.int32),
            pltpu.VMEM((_NFACE,), jnp.int32),
            pltpu.VMEM((_NFACE,), jnp.int32),
        ],
    )
    def sc_kernel(ci_hbm, segm_hbm, par_hbm, ep_hbm, out_hbm,
                  in_v, out_v, segm_v, par_v, ep_v, st_v, ef_v):
        wid = lax.axis_index("s") * jnp.int32(2) + lax.axis_index("c")
        start_g = jnp.minimum(wid * jnp.int32(gpw), jnp.int32(groups - gpw))
        base = start_g * jnp.int32(_GROUPW)
        pltpu.sync_copy(ci_hbm.at[pl.ds(base, chunk)], in_v)
        pltpu.sync_copy(segm_hbm, segm_v)
        pltpu.sync_copy(par_hbm, par_v)
        pltpu.sync_copy(ep_hbm, ep_v)

        lane4 = lax.iota(jnp.int32, _L) * jnp.int32(4)
        one = jnp.full((_L,), 1, jnp.int32)
        zero = jnp.full((_L,), 0, jnp.int32)
        neg1 = jnp.full((_L,), -1, jnp.int32)

        bits = []
        for k in range(n_extra):
            a = ep_v[pl.ds(2 * k * _L, _L)]
            b = ep_v[pl.ds((2 * k + 1) * _L, _L)]
            bits.append((a, b, one << a, one << b))

        for j in range(_NFACE // _L):
            sl = pl.ds(j * _L, _L)
            s = segm_v[sl]
            p = par_v[sl]
            st_v[sl] = s | (p << 7)
            ef = zero
            for a, b, bit_a, bit_b in bits:
                ef = ef | jnp.where(s == a, bit_b, zero)
                ef = ef | jnp.where(s == b, bit_a, zero)
            ef_v[sl] = ef

        @plsc.parallel_loop(jnp.int32(0), jnp.int32(chunk), jnp.int32(_GROUPW),
                            unroll=_UNROLL)
        def _loop(off):
            idx_a = lane4 + off
            idx_b = idx_a + jnp.int32(2)
            ia = plsc.load_gather(in_v, [idx_a])
            ib = plsc.load_gather(in_v, [idx_b])
            sta = plsc.load_gather(st_v, [ia])
            stb = plsc.load_gather(st_v, [ib])
            efa = plsc.load_gather(ef_v, [ia])
            s0 = sta & jnp.int32(127)
            p0 = sta >> jnp.int32(7)
            s1 = stb & jnp.int32(127)
            p1 = stb >> jnp.int32(7)
            t = ((s0 == s1) | (s0 == p1)) | (s1 == p0)
            t = t & (ia >= jnp.int32(0))
            m = t | (((efa >> s1) & one) != zero)
            plsc.store_scatter(out_v, [idx_a], jnp.where(m, neg1, ia))
            plsc.store_scatter(out_v, [idx_b], jnp.where(m, neg1, ib))

        lane2 = lax.iota(jnp.int32, _L) * jnp.int32(2)

        @plsc.parallel_loop(jnp.int32(0), jnp.int32(chunk), jnp.int32(2 * _L),
                            unroll=_UNROLL)
        def _loop_hi(off):
            lo = plsc.load_gather(out_v, [lane2 + off])
            plsc.store_scatter(out_v, [lane2 + off + one],
                               lo >> jnp.int32(31))

        pltpu.sync_copy(out_v, out_hbm.at[pl.ds(base, chunk)])

    return sc_kernel


def kernel(collision_idxs, faces_segm, faces_parents, extra_pairs):
    shape = collision_idxs.shape
    words = lax.bitcast_convert_type(collision_idxs, jnp.int32).reshape(-1)
    segm = faces_segm.astype(jnp.int32)
    par = faces_parents.astype(jnp.int32)
    n_extra = extra_pairs.shape[0]
    ep = jnp.repeat(extra_pairs.astype(jnp.int32).reshape(-1), _L)
    out = _build_sc_kernel(words.shape[0], n_extra)(words, segm, par, ep)
    return lax.bitcast_convert_type(out.reshape(shape + (2,)), jnp.int64)


# speedup vs baseline: 143.4783x; 15.1640x over previous
import functools

import jax
import jax.numpy as jnp
from jax import lax
from jax.experimental import pallas as pl
from jax.experimental.pallas import tpu as pltpu
from jax.experimental.pallas import tpu_sc as plsc

_L = 16
_NW = 32
_NFACE = 128
_UNROLL = 4


@functools.lru_cache(maxsize=None)
def _build_sc_kernel(n_pairs: int, n_extra: int):
    assert n_pairs % _L == 0
    groups = n_pairs // _L
    gpw = -(-groups // _NW)
    gpw = -(-gpw // _UNROLL) * _UNROLL
    ppw = gpw * _L
    mesh = plsc.VectorSubcoreMesh(core_axis_name="c", subcore_axis_name="s")
    plane = jax.ShapeDtypeStruct((n_pairs,), jnp.int32)

    @functools.partial(
        pl.kernel,
        mesh=mesh,
        out_type=(plane, plane, plane),
        compiler_params=pltpu.CompilerParams(needs_layout_passes=False),
        scratch_types=[
            pltpu.VMEM((ppw,), jnp.int32),
            pltpu.VMEM((ppw,), jnp.int32),
            pltpu.VMEM((ppw,), jnp.int32),
            pltpu.VMEM((ppw,), jnp.int32),
            pltpu.VMEM((ppw,), jnp.int32),
            pltpu.VMEM((_NFACE,), jnp.int32),
            pltpu.VMEM((_NFACE,), jnp.int32),
            pltpu.VMEM((2 * n_extra * _L,), jnp.int32),
            pltpu.VMEM((_NFACE,), jnp.int32),
            pltpu.VMEM((_NFACE,), jnp.int32),
        ],
    )
    def sc_kernel(a0_hbm, a1_hbm, segm_hbm, par_hbm, ep_hbm,
                  lo0_hbm, lo1_hbm, hi_hbm,
                  a0_v, a1_v, lo0_v, lo1_v, hi_v,
                  segm_v, par_v, ep_v, st_v, ef_v):
        wid = lax.axis_index("s") * jnp.int32(2) + lax.axis_index("c")
        start_g = jnp.minimum(wid * jnp.int32(gpw), jnp.int32(groups - gpw))
        base = start_g * jnp.int32(_L)
        pltpu.sync_copy(a0_hbm.at[pl.ds(base, ppw)], a0_v)
        pltpu.sync_copy(a1_hbm.at[pl.ds(base, ppw)], a1_v)
        pltpu.sync_copy(segm_hbm, segm_v)
        pltpu.sync_copy(par_hbm, par_v)
        pltpu.sync_copy(ep_hbm, ep_v)

        one = jnp.full((_L,), 1, jnp.int32)
        zero = jnp.full((_L,), 0, jnp.int32)
        neg1 = jnp.full((_L,), -1, jnp.int32)

        bits = []
        for k in range(n_extra):
            a = ep_v[pl.ds(2 * k * _L, _L)]
            b = ep_v[pl.ds((2 * k + 1) * _L, _L)]
            bits.append((a, b, one << a, one << b))

        for j in range(_NFACE // _L):
            sl = pl.ds(j * _L, _L)
            s = segm_v[sl]
            p = par_v[sl]
            st_v[sl] = s | (p << 7)
            ef = zero
            for a, b, bit_a, bit_b in bits:
                ef = ef | jnp.where(s == a, bit_b, zero)
                ef = ef | jnp.where(s == b, bit_a, zero)
            ef_v[sl] = ef

        @plsc.parallel_loop(jnp.int32(0), jnp.int32(ppw), jnp.int32(_L),
                            unroll=_UNROLL)
        def _loop(off):
            sl = pl.ds(off, _L)
            ia = a0_v[sl]
            ib = a1_v[sl]
            sta = plsc.load_gather(st_v, [ia])
            stb = plsc.load_gather(st_v, [ib])
            efa = plsc.load_gather(ef_v, [ia])
            s0 = sta & jnp.int32(127)
            p0 = sta >> jnp.int32(7)
            s1 = stb & jnp.int32(127)
            p1 = stb >> jnp.int32(7)
            t = ((s0 == s1) | (s0 == p1)) | (s1 == p0)
            t = t & (ia >= jnp.int32(0))
            m = t | (((efa >> s1) & one) != zero)
            lo0_v[sl] = jnp.where(m, neg1, ia)
            lo1_v[sl] = jnp.where(m, neg1, ib)
            hi_v[sl] = jnp.where(m, neg1, zero)

        pltpu.sync_copy(lo0_v, lo0_hbm.at[pl.ds(base, ppw)])
        pltpu.sync_copy(lo1_v, lo1_hbm.at[pl.ds(base, ppw)])
        pltpu.sync_copy(hi_v, hi_hbm.at[pl.ds(base, ppw)])

    return sc_kernel


def kernel(collision_idxs, faces_segm, faces_parents, extra_pairs):
    n_pairs = collision_idxs.shape[1]
    words = lax.bitcast_convert_type(collision_idxs, jnp.int32)
    a0 = words[0, :, 0, 0]
    a1 = words[0, :, 1, 0]
    segm = faces_segm.astype(jnp.int32)
    par = faces_parents.astype(jnp.int32)
    n_extra = extra_pairs.shape[0]
    ep = jnp.repeat(extra_pairs.astype(jnp.int32).reshape(-1), _L)
    lo0, lo1, hi = _build_sc_kernel(n_pairs, n_extra)(a0, a1, segm, par, ep)
    z = jnp.stack([jnp.stack([lo0, hi], axis=-1),
                   jnp.stack([lo1, hi], axis=-1)], axis=1)
    return lax.bitcast_convert_type(z[None], jnp.int64)
